# Initial kernel scaffold; baseline (speedup 1.0000x reference)
import jax
import jax.numpy as jnp
from jax.experimental import pallas as pl


def kernel(x, table, t_table, W1, w2):
    raise NotImplementedError("write your pallas kernel here")


# speedup vs baseline: 1.0922x; 1.0922x over previous
import functools

import jax
import jax.numpy as jnp
import numpy as np
from jax import lax
from jax.experimental import pallas as pl
from jax.experimental.pallas import tpu as pltpu
from jax.experimental.pallas import tpu_sc as plsc

_NUM_T = 49
_MASK_ID = 1024
_D = 256
_B = 128
_S = 128
_N = _B * _S
_TOK_PAD = 1152
_T_PAD = 56
_TBLK = 8
_NC = 2
_NS = 16
_NW = _NC * _NS
_EPW = _N // _NW
_BPW = _B // _NW
_NVEC = _EPW // 16

def _rng_consts():
    key = jax.random.key(42)
    kt, km = jax.random.split(key)
    t = jax.random.randint(kt, (_B,), 1, _NUM_T + 1, dtype=jnp.int32)
    t_mask = t.reshape(_B, 1, 1).astype(jnp.float32)
    u = jax.random.uniform(km, (_B, 1, _S), dtype=jnp.float32)
    mask = u < (t_mask / _NUM_T)
    return t, mask


def _g_table_body(table_ref, tcat_ref, w1_ref, w2_ref, o_ref, tw1_ref):
    @pl.when(pl.program_id(0) == 0)
    def _():
        tw1_ref[...] = jnp.dot(table_ref[...], w1_ref[...],
                               preferred_element_type=jnp.float32)

    tt1 = jnp.dot(tcat_ref[...], w1_ref[...],
                  preferred_element_type=jnp.float32)
    tw1 = tw1_ref[...]
    w2r = w2_ref[...]
    for j in range(_TBLK):
        h = jnp.maximum(tw1 + tt1[j:j + 1, :], 0.0)
        o_ref[j, :] = jnp.sum(h * w2r, axis=1)


def _build_g(table_pad, t_pad, w1, w2row):
    return pl.pallas_call(
        _g_table_body,
        grid=(_T_PAD // _TBLK,),
        in_specs=[
            pl.BlockSpec((_TOK_PAD, _D), lambda i: (0, 0)),
            pl.BlockSpec((_TBLK, _D), lambda i: (i, 0)),
            pl.BlockSpec((_D, _D), lambda i: (0, 0)),
            pl.BlockSpec((1, _D), lambda i: (0, 0)),
        ],
        out_specs=pl.BlockSpec((_TBLK, _TOK_PAD), lambda i: (i, 0)),
        out_shape=jax.ShapeDtypeStruct((_T_PAD, _TOK_PAD), jnp.float32),
        scratch_shapes=[pltpu.VMEM((_TOK_PAD, _D), jnp.float32)],
    )(table_pad, t_pad, w1, w2row)


def _sc_loss_body(g_hbm, x_hbm, m_hbm, tb_hbm, out_hbm,
                  gv, xv, mv, tbv, av, sem):
    wid = lax.axis_index("s") * _NC + lax.axis_index("c")
    base = wid * _EPW
    g_cp = pltpu.async_copy(g_hbm, gv, sem)
    pltpu.sync_copy(x_hbm.at[pl.ds(base, _EPW)], xv)
    pltpu.sync_copy(m_hbm.at[pl.ds(base, _EPW)], mv)
    pltpu.sync_copy(tb_hbm.at[pl.ds(base, _EPW)], tbv)
    g_cp.wait()
    acc = jnp.zeros((16,), jnp.float32)
    for i in range(_NVEC):
        xs = xv[pl.ds(i * 16, 16)]
        ms = mv[pl.ds(i * 16, 16)]
        masked = ms != 0
        idx = tbv[pl.ds(i * 16, 16)] + jnp.where(masked, _MASK_ID, xs)
        val = plsc.load_gather(gv, [idx])
        ign = jnp.where(masked, xs.astype(jnp.float32), jnp.float32(-1.0))
        acc = acc + jnp.abs(val - ign)
    av[...] = acc
    pltpu.sync_copy(av, out_hbm.at[wid])


def _sc_loss(g_flat, x_flat, mask_flat, tbase_flat):
    mesh = plsc.VectorSubcoreMesh(core_axis_name="c", subcore_axis_name="s")
    run = functools.partial(
        pl.kernel,
        out_type=jax.ShapeDtypeStruct((_NW, 16), jnp.float32),
        mesh=mesh,
        compiler_params=pltpu.CompilerParams(needs_layout_passes=False),
        scratch_types=[
            pltpu.VMEM((_T_PAD * _TOK_PAD,), jnp.float32),
            pltpu.VMEM((_EPW,), jnp.int32),
            pltpu.VMEM((_EPW,), jnp.int32),
            pltpu.VMEM((_EPW,), jnp.int32),
            pltpu.VMEM((16,), jnp.float32),
            pltpu.SemaphoreType.DMA,
        ],
    )(_sc_loss_body)
    return run(g_flat, x_flat, mask_flat, tbase_flat)


def kernel(x, table, t_table, W1, w2):
    t, mask = _rng_consts()
    r = t - 1
    tbase_flat = jnp.repeat(r * _TOK_PAD, _S)
    mask_flat = mask.reshape(_N).astype(jnp.int32)

    table_pad = jnp.pad(table, ((0, _TOK_PAD - table.shape[0]), (0, 0)))
    t_pad = jnp.pad(t_table, ((0, _T_PAD - t_table.shape[0]), (0, 0)))
    w2row = w2.reshape(1, _D)
    g = _build_g(table_pad, t_pad, W1, w2row)

    x_flat = x.reshape(_N)
    partials = _sc_loss(g.reshape(_T_PAD * _TOK_PAD), x_flat, mask_flat,
                        tbase_flat)
    return jnp.sum(partials) / np.float32(_N)


# speedup vs baseline: 1.3738x; 1.2578x over previous
import functools

import jax
import jax.numpy as jnp
import numpy as np
from jax import lax
from jax.experimental import pallas as pl
from jax.experimental.pallas import tpu as pltpu
from jax.experimental.pallas import tpu_sc as plsc

_NUM_T = 49
_MASK_ID = 1024
_D = 256
_B = 128
_S = 128
_N = _B * _S
_TOK_PAD = 1152
_T_PAD = 56
_TBLK = 8
_NC = 2
_NS = 16
_NW = _NC * _NS
_EPW = _N // _NW
_BPW = _B // _NW
_NVEC = _EPW // 16

def _rng_consts():
    key = jax.random.key(42)
    kt, km = jax.random.split(key)
    t = jax.random.randint(kt, (_B,), 1, _NUM_T + 1, dtype=jnp.int32)
    t_mask = t.reshape(_B, 1, 1).astype(jnp.float32)
    u = jax.random.uniform(km, (_B, 1, _S), dtype=jnp.float32)
    mask = u < (t_mask / _NUM_T)
    return t, mask


_CAT = _TOK_PAD + _T_PAD


def _g_mm_body(tcat_ref, w1_ref, o_ref):
    o_ref[...] = jnp.dot(tcat_ref[...], w1_ref[...],
                         preferred_element_type=jnp.float32)


def _g_table_body(mm_ref, w2_ref, o_ref, tw1t_ref):
    @pl.when(pl.program_id(0) == 0)
    def _():
        tw1t_ref[...] = mm_ref[:_TOK_PAD, :].T

    i = pl.program_id(0)
    tw1t = tw1t_ref[...]
    tt1t = mm_ref[pl.ds(_TOK_PAD + i * _TBLK, _TBLK), :].T
    w2c = w2_ref[...]
    for j in range(_TBLK):
        h = jnp.maximum(tw1t + tt1t[:, j:j + 1], 0.0) * w2c
        o_ref[j, :] = jnp.sum(h, axis=0)


def _build_g(tcat, w1, w2):
    mm = pl.pallas_call(
        _g_mm_body,
        out_shape=jax.ShapeDtypeStruct((_CAT, _D), jnp.float32),
    )(tcat, w1)
    return pl.pallas_call(
        _g_table_body,
        grid=(_T_PAD // _TBLK,),
        in_specs=[
            pl.BlockSpec((_CAT, _D), lambda i: (0, 0)),
            pl.BlockSpec((_D, 1), lambda i: (0, 0)),
        ],
        out_specs=pl.BlockSpec((_TBLK, _TOK_PAD), lambda i: (i, 0)),
        out_shape=jax.ShapeDtypeStruct((_T_PAD, _TOK_PAD), jnp.float32),
        scratch_shapes=[pltpu.VMEM((_D, _TOK_PAD), jnp.float32)],
    )(mm, w2)


def _sc_loss_body(g_hbm, x_hbm, m_hbm, tb_hbm, out_hbm,
                  gv, xv, mv, tbv, av, sem):
    wid = lax.axis_index("s") * _NC + lax.axis_index("c")
    base = wid * _EPW
    g_cp = pltpu.async_copy(g_hbm, gv, sem)
    pltpu.sync_copy(x_hbm.at[pl.ds(base, _EPW)], xv)
    pltpu.sync_copy(m_hbm.at[pl.ds(base, _EPW)], mv)
    pltpu.sync_copy(tb_hbm.at[pl.ds(base, _EPW)], tbv)
    g_cp.wait()
    acc = jnp.zeros((16,), jnp.float32)
    for i in range(_NVEC):
        xs = xv[pl.ds(i * 16, 16)]
        ms = mv[pl.ds(i * 16, 16)]
        masked = ms != 0
        idx = tbv[pl.ds(i * 16, 16)] + jnp.where(masked, _MASK_ID, xs)
        val = plsc.load_gather(gv, [idx])
        ign = jnp.where(masked, xs.astype(jnp.float32), jnp.float32(-1.0))
        acc = acc + jnp.abs(val - ign)
    av[...] = acc
    pltpu.sync_copy(av, out_hbm.at[wid])


def _sc_loss(g_flat, x_flat, mask_flat, tbase_flat):
    mesh = plsc.VectorSubcoreMesh(core_axis_name="c", subcore_axis_name="s")
    run = functools.partial(
        pl.kernel,
        out_type=jax.ShapeDtypeStruct((_NW, 16), jnp.float32),
        mesh=mesh,
        compiler_params=pltpu.CompilerParams(needs_layout_passes=False),
        scratch_types=[
            pltpu.VMEM((_T_PAD * _TOK_PAD,), jnp.float32),
            pltpu.VMEM((_EPW,), jnp.int32),
            pltpu.VMEM((_EPW,), jnp.int32),
            pltpu.VMEM((_EPW,), jnp.int32),
            pltpu.VMEM((16,), jnp.float32),
            pltpu.SemaphoreType.DMA,
        ],
    )(_sc_loss_body)
    return run(g_flat, x_flat, mask_flat, tbase_flat)


def kernel(x, table, t_table, W1, w2):
    t, mask = _rng_consts()
    r = t - 1
    tbase_flat = jnp.repeat(r * _TOK_PAD, _S)
    mask_flat = mask.reshape(_N).astype(jnp.int32)

    tcat = jnp.concatenate([
        jnp.pad(table, ((0, _TOK_PAD - table.shape[0]), (0, 0))),
        jnp.pad(t_table, ((0, _T_PAD - t_table.shape[0]), (0, 0))),
    ])
    g = _build_g(tcat, W1, w2)

    x_flat = x.reshape(_N)
    partials = _sc_loss(g.reshape(_T_PAD * _TOK_PAD), x_flat, mask_flat,
                        tbase_flat)
    return jnp.sum(partials) / np.float32(_N)


# speedup vs baseline: 2.3110x; 1.6822x over previous
import functools

import jax
import jax.numpy as jnp
import numpy as np
from jax import lax
from jax.experimental import pallas as pl
from jax.experimental.pallas import tpu as pltpu
from jax.experimental.pallas import tpu_sc as plsc

_NUM_T = 49
_MASK_ID = 1024
_D = 256
_B = 128
_S = 128
_N = _B * _S
_NTOK = _MASK_ID + 1
_TOK_PAD = 1152
_T_PAD = 56
_TBLK = 8
_CAT = _TOK_PAD + _T_PAD
_NC = 2
_NS = 16
_NW = _NC * _NS
_EPW = _N // _NW
_BPW = _B // _NW
_NVEC = _EPW // 16


def _tf_rounds(x0, x1, rots):
    for r in rots:
        x0 = (x0 + x1).astype(np.uint32)
        x1 = ((x1 << np.uint32(r)) | (x1 >> np.uint32(32 - r))).astype(np.uint32)
        x1 = (x0 ^ x1).astype(np.uint32)
    return x0, x1


def _threefry2x32(k1, k2, x0, x1):
    r1 = (13, 15, 26, 6)
    r2 = (17, 29, 16, 24)
    ks0 = np.uint32(k1)
    ks1 = np.uint32(k2)
    ks2 = np.uint32(ks0 ^ ks1 ^ np.uint32(0x1BD11BDA))
    x0 = (x0 + ks0).astype(np.uint32)
    x1 = (x1 + ks1).astype(np.uint32)
    sched = [(r1, ks1, ks2), (r2, ks2, ks0), (r1, ks0, ks1),
             (r2, ks1, ks2), (r1, ks2, ks0)]
    for i, (rots, a, b) in enumerate(sched):
        x0, x1 = _tf_rounds(x0, x1, rots)
        x0 = (x0 + a).astype(np.uint32)
        x1 = (x1 + b + np.uint32(i + 1)).astype(np.uint32)
    return x0, x1


def _random_bits32(k1, k2, n):
    b1, b2 = _threefry2x32(k1, k2, np.zeros(n, np.uint32),
                           np.arange(n, dtype=np.uint32))
    return (b1 ^ b2).astype(np.uint32)


def _rng_consts():
    k1, k2 = np.uint32(0), np.uint32(42)
    sb1, sb2 = _threefry2x32(k1, k2, np.zeros(2, np.uint32),
                             np.arange(2, dtype=np.uint32))
    kt = (sb1[0], sb2[0])
    km = (sb1[1], sb2[1])
    rb1, rb2 = _threefry2x32(kt[0], kt[1], np.zeros(2, np.uint32),
                             np.arange(2, dtype=np.uint32))
    higher = _random_bits32(rb1[0], rb2[0], _B)
    lower = _random_bits32(rb1[1], rb2[1], _B)
    span = np.uint32(_NUM_T)
    mult = np.uint32((np.uint64(np.uint32(65536) % span) *
                      np.uint64(np.uint32(65536) % span)) % span)
    unbiased = ((higher % span) * mult + (lower % span)) % span
    t = (1 + unbiased.astype(np.int32)).astype(np.int32)
    ubits = _random_bits32(km[0], km[1], _N)
    fbits = ((ubits >> np.uint32(9)) | np.uint32(0x3F800000)).view(np.float32)
    u = (fbits - np.float32(1.0)).reshape(_B, _S)
    mask = u < (t.astype(np.float32).reshape(_B, 1) / np.float32(_NUM_T))
    return t, mask


def _g_table_body(table_ref, tt_ref, w1_ref, w2_ref, o_ref, mm_ref, tw1t_ref):
    @pl.when(pl.program_id(0) == 0)
    def _():
        w1v = w1_ref[...]
        mm_ref[pl.ds(0, _NTOK), :] = jnp.dot(
            table_ref[...], w1v, preferred_element_type=jnp.float32)
        mm_ref[pl.ds(_TOK_PAD, _NUM_T), :] = jnp.dot(
            tt_ref[...], w1v, preferred_element_type=jnp.float32)
        tw1t_ref[...] = mm_ref[:_TOK_PAD, :].T

    i = pl.program_id(0)
    tw1t = tw1t_ref[...]
    tt1t = mm_ref[pl.ds(_TOK_PAD + i * _TBLK, _TBLK), :].T
    w2c = w2_ref[...]
    for j in range(_TBLK):
        h = jnp.maximum(tw1t + tt1t[:, j:j + 1], 0.0) * w2c
        o_ref[j, :] = jnp.sum(h, axis=0)


def _build_g(table, t_table, w1, w2):
    return pl.pallas_call(
        _g_table_body,
        grid=(_T_PAD // _TBLK,),
        in_specs=[
            pl.BlockSpec((_NTOK, _D), lambda i: (0, 0)),
            pl.BlockSpec((_NUM_T, _D), lambda i: (0, 0)),
            pl.BlockSpec((_D, _D), lambda i: (0, 0)),
            pl.BlockSpec((_D, 1), lambda i: (0, 0)),
        ],
        out_specs=pl.BlockSpec((_TBLK, _TOK_PAD), lambda i: (i, 0)),
        out_shape=jax.ShapeDtypeStruct((_T_PAD, _TOK_PAD), jnp.float32),
        scratch_shapes=[pltpu.VMEM((_CAT, _D), jnp.float32),
                        pltpu.VMEM((_D, _TOK_PAD), jnp.float32)],
    )(table, t_table, w1, w2)


def _sc_loss_body(g_hbm, x_hbm, m_hbm, rows_hbm, out_hbm,
                  gv, xv, mv, rv, av, sem, sem_x, sem_m):
    wid = lax.axis_index("s") * _NC + lax.axis_index("c")
    x_cp = pltpu.async_copy(x_hbm.at[pl.ds(wid * _BPW, _BPW)], xv, sem_x)
    m_cp = pltpu.async_copy(m_hbm.at[wid], mv, sem_m)
    pltpu.sync_copy(rows_hbm.at[wid], rv)
    g_cp = pltpu.async_copy(g_hbm.at[rv], gv, sem)
    x_cp.wait()
    m_cp.wait()
    g_cp.wait()
    acc = jnp.zeros((16,), jnp.float32)
    for i in range(_NVEC):
        xs = xv[i // (_S // 16), 0, pl.ds((i % (_S // 16)) * 16, 16)]
        ms = mv[pl.ds(i * 16, 16)]
        masked = ms != 0
        row = jnp.full((16,), i // (_S // 16), jnp.int32)
        col = jnp.where(masked, _MASK_ID, xs)
        val = plsc.load_gather(gv, [row, col])
        ign = jnp.where(masked, xs.astype(jnp.float32), jnp.float32(-1.0))
        acc = acc + jnp.abs(val - ign)
    av[...] = acc
    pltpu.sync_copy(av, out_hbm.at[wid])


def _sc_loss(g, x, mask_c, rows_c):
    mesh = plsc.VectorSubcoreMesh(core_axis_name="c", subcore_axis_name="s")
    run = functools.partial(
        pl.kernel,
        out_type=jax.ShapeDtypeStruct((_NW, 16), jnp.float32),
        mesh=mesh,
        compiler_params=pltpu.CompilerParams(needs_layout_passes=False),
        scratch_types=[
            pltpu.VMEM((8, _TOK_PAD), jnp.float32),
            pltpu.VMEM((_BPW, 1, _S), jnp.int32),
            pltpu.VMEM((_EPW,), jnp.int32),
            pltpu.VMEM((8,), jnp.int32),
            pltpu.VMEM((16,), jnp.float32),
            pltpu.SemaphoreType.DMA,
            pltpu.SemaphoreType.DMA,
            pltpu.SemaphoreType.DMA,
        ],
    )(_sc_loss_body)
    return run(g, x, mask_c, rows_c)


def kernel(x, table, t_table, W1, w2):
    t_np, mask_np = _rng_consts()
    r_np = (t_np - 1).astype(np.int32)
    rows_c = np.zeros((_NW, 8), np.int32)
    rows_c[:, :_BPW] = r_np.reshape(_NW, _BPW)
    mask_c = mask_np.reshape(_NW, _EPW).astype(np.int32)

    g = _build_g(table, t_table, W1, w2)
    partials = _sc_loss(g, x, mask_c, rows_c)
    return jnp.sum(partials) / np.float32(_N)


# speedup vs baseline: 2.6897x; 1.1639x over previous
import functools

import jax
import jax.numpy as jnp
import numpy as np
from jax import lax
from jax.experimental import pallas as pl
from jax.experimental.pallas import tpu as pltpu
from jax.experimental.pallas import tpu_sc as plsc

_NUM_T = 49
_MASK_ID = 1024
_D = 256
_B = 128
_S = 128
_N = _B * _S
_NTOK = _MASK_ID + 1
_TOK_PAD = 1152
_T_PAD = 56
_TBLK = 8
_CAT = _TOK_PAD + _T_PAD
_NC = 1
_NS = 16
_NW = _NC * _NS
_EPW = _N // _NW
_BPW = _B // _NW
_NVEC = _EPW // 16


def _tf_rounds(x0, x1, rots):
    for r in rots:
        x0 = (x0 + x1).astype(np.uint32)
        x1 = ((x1 << np.uint32(r)) | (x1 >> np.uint32(32 - r))).astype(np.uint32)
        x1 = (x0 ^ x1).astype(np.uint32)
    return x0, x1


def _threefry2x32(k1, k2, x0, x1):
    r1 = (13, 15, 26, 6)
    r2 = (17, 29, 16, 24)
    ks0 = np.uint32(k1)
    ks1 = np.uint32(k2)
    ks2 = np.uint32(ks0 ^ ks1 ^ np.uint32(0x1BD11BDA))
    x0 = (x0 + ks0).astype(np.uint32)
    x1 = (x1 + ks1).astype(np.uint32)
    sched = [(r1, ks1, ks2), (r2, ks2, ks0), (r1, ks0, ks1),
             (r2, ks1, ks2), (r1, ks2, ks0)]
    for i, (rots, a, b) in enumerate(sched):
        x0, x1 = _tf_rounds(x0, x1, rots)
        x0 = (x0 + a).astype(np.uint32)
        x1 = (x1 + b + np.uint32(i + 1)).astype(np.uint32)
    return x0, x1


def _random_bits32(k1, k2, n):
    b1, b2 = _threefry2x32(k1, k2, np.zeros(n, np.uint32),
                           np.arange(n, dtype=np.uint32))
    return (b1 ^ b2).astype(np.uint32)


def _rng_consts():
    k1, k2 = np.uint32(0), np.uint32(42)
    sb1, sb2 = _threefry2x32(k1, k2, np.zeros(2, np.uint32),
                             np.arange(2, dtype=np.uint32))
    kt = (sb1[0], sb2[0])
    km = (sb1[1], sb2[1])
    rb1, rb2 = _threefry2x32(kt[0], kt[1], np.zeros(2, np.uint32),
                             np.arange(2, dtype=np.uint32))
    higher = _random_bits32(rb1[0], rb2[0], _B)
    lower = _random_bits32(rb1[1], rb2[1], _B)
    span = np.uint32(_NUM_T)
    mult = np.uint32((np.uint64(np.uint32(65536) % span) *
                      np.uint64(np.uint32(65536) % span)) % span)
    unbiased = ((higher % span) * mult + (lower % span)) % span
    t = (1 + unbiased.astype(np.int32)).astype(np.int32)
    ubits = _random_bits32(km[0], km[1], _N)
    fbits = ((ubits >> np.uint32(9)) | np.uint32(0x3F800000)).view(np.float32)
    u = (fbits - np.float32(1.0)).reshape(_B, _S)
    mask = u < (t.astype(np.float32).reshape(_B, 1) / np.float32(_NUM_T))
    return t, mask


def _g_table_body(table_ref, tt_ref, w1_ref, w2_ref, o_ref, mm_ref, tw1t_ref):
    @pl.when(pl.program_id(0) == 0)
    def _():
        w1v = w1_ref[...]
        mm_ref[pl.ds(0, _NTOK), :] = jnp.dot(
            table_ref[...], w1v, preferred_element_type=jnp.float32)
        mm_ref[pl.ds(_TOK_PAD, _NUM_T), :] = jnp.dot(
            tt_ref[...], w1v, preferred_element_type=jnp.float32)
        tw1t_ref[...] = mm_ref[:_TOK_PAD, :].T

    i = pl.program_id(0)
    tw1t = tw1t_ref[...]
    tt1t = mm_ref[pl.ds(_TOK_PAD + i * _TBLK, _TBLK), :].T
    w2c = w2_ref[...]
    for j in range(_TBLK):
        h = jnp.maximum(tw1t + tt1t[:, j:j + 1], 0.0) * w2c
        o_ref[j, :] = jnp.sum(h, axis=0)


def _build_g(table, t_table, w1, w2):
    return pl.pallas_call(
        _g_table_body,
        grid=(_T_PAD // _TBLK,),
        in_specs=[
            pl.BlockSpec((_NTOK, _D), lambda i: (0, 0)),
            pl.BlockSpec((_NUM_T, _D), lambda i: (0, 0)),
            pl.BlockSpec((_D, _D), lambda i: (0, 0)),
            pl.BlockSpec((_D, 1), lambda i: (0, 0)),
        ],
        out_specs=pl.BlockSpec((_TBLK, _TOK_PAD), lambda i: (i, 0)),
        out_shape=jax.ShapeDtypeStruct((_T_PAD, _TOK_PAD), jnp.float32),
        scratch_shapes=[pltpu.VMEM((_CAT, _D), jnp.float32),
                        pltpu.VMEM((_D, _TOK_PAD), jnp.float32)],
    )(table, t_table, w1, w2)


def _sc_loss_body(g_hbm, x_hbm, m_hbm, rows_hbm, out_hbm,
                  gv, xv, mv, rv, av, sem, sem_x, sem_m):
    wid = lax.axis_index("s") * _NC + lax.axis_index("c")
    x_cp = pltpu.async_copy(x_hbm.at[pl.ds(wid * _BPW, _BPW)], xv, sem_x)
    m_cp = pltpu.async_copy(m_hbm.at[wid], mv, sem_m)
    pltpu.sync_copy(rows_hbm.at[wid], rv)
    g_cp = pltpu.async_copy(g_hbm.at[rv], gv, sem)
    x_cp.wait()
    m_cp.wait()
    g_cp.wait()
    acc = jnp.zeros((16,), jnp.float32)
    for i in range(_NVEC):
        xs = xv[i // (_S // 16), 0, pl.ds((i % (_S // 16)) * 16, 16)]
        ms = mv[pl.ds(i * 16, 16)]
        masked = ms != 0
        row = jnp.full((16,), i // (_S // 16), jnp.int32)
        col = jnp.where(masked, _MASK_ID, xs)
        val = plsc.load_gather(gv, [row, col])
        ign = jnp.where(masked, xs.astype(jnp.float32), jnp.float32(-1.0))
        acc = acc + jnp.abs(val - ign)
    av[...] = acc
    pltpu.sync_copy(av, out_hbm.at[wid])


def _sc_loss(g, x, mask_c, rows_c):
    mesh = plsc.VectorSubcoreMesh(core_axis_name="c", subcore_axis_name="s", num_cores=1)
    run = functools.partial(
        pl.kernel,
        out_type=jax.ShapeDtypeStruct((_NW, 16), jnp.float32),
        mesh=mesh,
        compiler_params=pltpu.CompilerParams(needs_layout_passes=False),
        scratch_types=[
            pltpu.VMEM((8, _TOK_PAD), jnp.float32),
            pltpu.VMEM((_BPW, 1, _S), jnp.int32),
            pltpu.VMEM((_EPW,), jnp.int32),
            pltpu.VMEM((8,), jnp.int32),
            pltpu.VMEM((16,), jnp.float32),
            pltpu.SemaphoreType.DMA,
            pltpu.SemaphoreType.DMA,
            pltpu.SemaphoreType.DMA,
        ],
    )(_sc_loss_body)
    return run(g, x, mask_c, rows_c)


def kernel(x, table, t_table, W1, w2):
    t_np, mask_np = _rng_consts()
    r_np = (t_np - 1).astype(np.int32)
    rows_c = np.zeros((_NW, 8), np.int32)
    rows_c[:, :_BPW] = r_np.reshape(_NW, _BPW)
    mask_c = mask_np.reshape(_NW, _EPW).astype(np.int32)

    g = _build_g(table, t_table, W1, w2)
    partials = _sc_loss(g, x, mask_c, rows_c)
    return jnp.sum(partials) / np.float32(_N)
